# baseline (speedup 1.0000x reference)
import functools

import jax
import jax.numpy as jnp
from jax import lax
from jax.experimental import pallas as pl
from jax.experimental.pallas import tpu as pltpu
from jax.experimental.pallas import tpu_sc as plsc

_B = 16384
_EMB = 64
_NC = 2
_NS = 16
_NW = _NC * _NS
_CHUNK = _B // _NW
_NGROUP = _CHUNK // 16
_GCHUNK = 128
_NGC = _CHUNK // _GCHUNK

_UV0, _UV1, _UV2, _UV3 = 944, 22, 2, 5
_GV0, _GV1, _GV2 = 1683, 5, 20
_NCU = _UV1 * _UV2 * _UV3
_NCG = _GV1 * _GV2


def _body(gid_h, pt_h, cat_h, uid_h, job_h, sex_h, age_h,
          wu0_h, wg0_h, cu_h, cg_h, w1g0_h, w1c_h, out_h,
          gidb, uidb, ptb, catb, jobb, sexb, ageb,
          ubuf, gbuf, cuv, cgv, w1g0v, w1cv, outb, gsems, rsem):
    wid = lax.axis_index("s") * _NC + lax.axis_index("c")
    base = wid * _CHUNK

    pltpu.sync_copy(gid_h.at[pl.ds(base, _CHUNK)], gidb)
    pltpu.sync_copy(uid_h.at[pl.ds(base, _CHUNK)], uidb)
    copies = []
    for c in range(_NGC):
        sl = pl.ds(c * _GCHUNK, _GCHUNK)
        copies.append(pltpu.async_copy(
            wg0_h.at[gidb.at[sl]], gbuf.at[sl], gsems.at[c]))
        copies.append(pltpu.async_copy(
            wu0_h.at[uidb.at[sl]], ubuf.at[sl], gsems.at[c]))
    rcopies = [
        pltpu.async_copy(pt_h.at[pl.ds(base, _CHUNK)], ptb, rsem),
        pltpu.async_copy(cat_h.at[pl.ds(base, _CHUNK)], catb, rsem),
        pltpu.async_copy(job_h.at[pl.ds(base, _CHUNK)], jobb, rsem),
        pltpu.async_copy(sex_h.at[pl.ds(base, _CHUNK)], sexb, rsem),
        pltpu.async_copy(age_h.at[pl.ds(base, _CHUNK)], ageb, rsem),
        pltpu.async_copy(cu_h, cuv, rsem),
        pltpu.async_copy(cg_h, cgv, rsem),
        pltpu.async_copy(w1g0_h, w1g0v, rsem),
        pltpu.async_copy(w1c_h, w1cv, rsem),
    ]
    for cp in rcopies:
        cp.wait()

    rowi = lax.iota(jnp.int32, 16)

    def group(g, carry):
        s0 = g * 16
        sl = pl.ds(s0, 16)
        gv = gidb[sl]
        pv = ptb[sl]
        cv = catb[sl]
        jv = jobb[sl]
        sv = sexb[sl]
        av = ageb[sl]
        cu = jv * (_UV2 * _UV3) + sv * _UV3 + av
        cg = pv * _GV2 + cv
        g1 = plsc.load_gather(w1g0v, [gv]) + plsc.load_gather(w1cv, [cg])
        cub = cu * _EMB
        cgb = cg * _EMB
        row = rowi + s0
        accs = [jnp.zeros((16,), jnp.float32) for _ in range(4)]
        for d in range(_EMB):
            coloff = (rowi + d) & (_EMB - 1)
            u = (plsc.load_gather(ubuf, [row, coloff])
                 + plsc.load_gather(cuv, [cub + coloff]))
            gg = (plsc.load_gather(gbuf, [row, coloff])
                  + plsc.load_gather(cgv, [cgb + coloff]))
            accs[d % 4] = accs[d % 4] + u * gg
        acc = (accs[0] + accs[1]) + (accs[2] + accs[3])
        logit = g1 + acc * (1.0 / 12.0)
        outb[sl] = 5.0 / (1.0 + jnp.exp(-logit))
        return carry

    for c in range(_NGC):
        copies[2 * c].wait()
        copies[2 * c + 1].wait()
        lax.fori_loop(c * (_NGROUP // _NGC), (c + 1) * (_NGROUP // _NGC),
                      group, 0)
    pltpu.sync_copy(outb, out_h.at[pl.ds(base, _CHUNK)])


@functools.cache
def _build_fm():
    mesh = plsc.VectorSubcoreMesh(
        core_axis_name="c", subcore_axis_name="s",
        num_cores=_NC, num_subcores=_NS)
    return pl.kernel(
        _body,
        out_type=jax.ShapeDtypeStruct((_B,), jnp.float32),
        mesh=mesh,
        compiler_params=pltpu.CompilerParams(
            needs_layout_passes=False, use_tc_tiling_on_sc=False),
        scratch_types=[
            pltpu.VMEM((_CHUNK,), jnp.int32),
            pltpu.VMEM((_CHUNK,), jnp.int32),
            pltpu.VMEM((_CHUNK,), jnp.int32),
            pltpu.VMEM((_CHUNK,), jnp.int32),
            pltpu.VMEM((_CHUNK,), jnp.int32),
            pltpu.VMEM((_CHUNK,), jnp.int32),
            pltpu.VMEM((_CHUNK,), jnp.int32),
            pltpu.VMEM((_CHUNK, _EMB), jnp.float32),
            pltpu.VMEM((_CHUNK, _EMB), jnp.float32),
            pltpu.VMEM((_NCU * _EMB,), jnp.float32),
            pltpu.VMEM((_NCG * _EMB,), jnp.float32),
            pltpu.VMEM((_GV0,), jnp.float32),
            pltpu.VMEM((_NCG,), jnp.float32),
            pltpu.VMEM((_CHUNK,), jnp.float32),
            pltpu.SemaphoreType.DMA((_NGC,)),
            pltpu.SemaphoreType.DMA,
        ],
    )


@jax.jit
def kernel(gid, pubtime, category, uid, job, sex, age,
           W1g0, W1g1, W1g2, Wu0, Wu1, Wu2, Wu3, Wg0, Wg1, Wg2):
    i32 = jnp.int32
    cuf = (Wu1[:, None, None, :] + Wu2[None, :, None, :]
           + Wu3[None, None, :, :]).reshape(_NCU * _EMB)
    cgf = (Wg1[:, None, :] + Wg2[None, :, :]).reshape(_NCG * _EMB)
    w1g0f = W1g0[:, 0]
    w1cf = (W1g1[:, 0][:, None] + W1g2[:, 0][None, :]).reshape(_NCG)

    out = _build_fm()(
        gid.astype(i32), pubtime.astype(i32), category.astype(i32),
        uid.astype(i32), job.astype(i32), sex.astype(i32), age.astype(i32),
        Wu0, Wg0, cuf, cgf, w1g0f, w1cf)
    return out[:, None]


# speedup vs baseline: 1.3357x; 1.3357x over previous
import functools

import jax
import jax.numpy as jnp
from jax import lax
from jax.experimental import pallas as pl
from jax.experimental.pallas import tpu as pltpu
from jax.experimental.pallas import tpu_sc as plsc

_B = 16384
_EMB = 64
_NC = 2
_NS = 16
_NW = _NC * _NS
_CHUNK = _B // _NW
_NGROUP = _CHUNK // 16
_GCHUNK = 128
_NGC = _CHUNK // _GCHUNK

_UV0, _UV1, _UV2, _UV3 = 944, 22, 2, 5
_GV0, _GV1, _GV2 = 1683, 5, 20
_NCU = _UV1 * _UV2 * _UV3
_NCG = _GV1 * _GV2


def _body(gid_h, pt_h, cat_h, uid_h, job_h, sex_h, age_h,
          wu0_h, wg0_h, cu_h, cg_h, w1g0_h, w1c_h, out_h,
          gidb, uidb, ptb, catb, jobb, sexb, ageb,
          ubuf, gbuf, cuv, cgv, w1g0v, w1cv, outb, gsems, rsem):
    wid = lax.axis_index("s") * _NC + lax.axis_index("c")
    base = wid * _CHUNK

    pltpu.sync_copy(gid_h.at[pl.ds(base, _CHUNK)], gidb)
    pltpu.sync_copy(uid_h.at[pl.ds(base, _CHUNK)], uidb)
    copies = []
    for c in range(_NGC):
        sl = pl.ds(c * _GCHUNK, _GCHUNK)
        copies.append(pltpu.async_copy(
            wg0_h.at[gidb.at[sl]], gbuf.at[sl], gsems.at[c]))
        copies.append(pltpu.async_copy(
            wu0_h.at[uidb.at[sl]], ubuf.at[sl], gsems.at[c]))
    rcopies = [
        pltpu.async_copy(pt_h.at[pl.ds(base, _CHUNK)], ptb, rsem),
        pltpu.async_copy(cat_h.at[pl.ds(base, _CHUNK)], catb, rsem),
        pltpu.async_copy(job_h.at[pl.ds(base, _CHUNK)], jobb, rsem),
        pltpu.async_copy(sex_h.at[pl.ds(base, _CHUNK)], sexb, rsem),
        pltpu.async_copy(age_h.at[pl.ds(base, _CHUNK)], ageb, rsem),
        pltpu.async_copy(cu_h, cuv, rsem),
        pltpu.async_copy(cg_h, cgv, rsem),
        pltpu.async_copy(w1g0_h, w1g0v, rsem),
        pltpu.async_copy(w1c_h, w1cv, rsem),
    ]
    for cp in rcopies:
        cp.wait()
    for cp in copies:
        cp.wait()

    rowi = lax.iota(jnp.int32, 16)

    def group(g, carry):
        s0 = g * 16
        sl = pl.ds(s0, 16)
        gv = gidb[sl]
        pv = ptb[sl]
        cv = catb[sl]
        jv = jobb[sl]
        sv = sexb[sl]
        av = ageb[sl]
        cu = jv * (_UV2 * _UV3) + sv * _UV3 + av
        cg = pv * _GV2 + cv
        g1 = plsc.load_gather(w1g0v, [gv]) + plsc.load_gather(w1cv, [cg])
        cub = cu * _EMB
        cgb = cg * _EMB
        row = rowi + s0
        accs = [jnp.zeros((16,), jnp.float32) for _ in range(4)]
        for d in range(_EMB):
            coloff = (rowi + d) & (_EMB - 1)
            u = (plsc.load_gather(ubuf, [row, coloff])
                 + plsc.load_gather(cuv, [cub + coloff]))
            gg = (plsc.load_gather(gbuf, [row, coloff])
                  + plsc.load_gather(cgv, [cgb + coloff]))
            accs[d % 4] = accs[d % 4] + u * gg
        acc = (accs[0] + accs[1]) + (accs[2] + accs[3])
        logit = g1 + acc * (1.0 / 12.0)
        outb[sl] = 5.0 / (1.0 + jnp.exp(-logit))
        return carry

    lax.fori_loop(0, _NGROUP, group, 0)
    pltpu.sync_copy(outb, out_h.at[pl.ds(base, _CHUNK)])


@functools.cache
def _build_fm():
    mesh = plsc.VectorSubcoreMesh(
        core_axis_name="c", subcore_axis_name="s",
        num_cores=_NC, num_subcores=_NS)
    return pl.kernel(
        _body,
        out_type=jax.ShapeDtypeStruct((_B,), jnp.float32),
        mesh=mesh,
        compiler_params=pltpu.CompilerParams(
            needs_layout_passes=False, use_tc_tiling_on_sc=False),
        scratch_types=[
            pltpu.VMEM((_CHUNK,), jnp.int32),
            pltpu.VMEM((_CHUNK,), jnp.int32),
            pltpu.VMEM((_CHUNK,), jnp.int32),
            pltpu.VMEM((_CHUNK,), jnp.int32),
            pltpu.VMEM((_CHUNK,), jnp.int32),
            pltpu.VMEM((_CHUNK,), jnp.int32),
            pltpu.VMEM((_CHUNK,), jnp.int32),
            pltpu.VMEM((_CHUNK, _EMB), jnp.float32),
            pltpu.VMEM((_CHUNK, _EMB), jnp.float32),
            pltpu.VMEM((_NCU * _EMB,), jnp.float32),
            pltpu.VMEM((_NCG * _EMB,), jnp.float32),
            pltpu.VMEM((_GV0,), jnp.float32),
            pltpu.VMEM((_NCG,), jnp.float32),
            pltpu.VMEM((_CHUNK,), jnp.float32),
            pltpu.SemaphoreType.DMA((_NGC,)),
            pltpu.SemaphoreType.DMA,
        ],
    )


@jax.jit
def kernel(gid, pubtime, category, uid, job, sex, age,
           W1g0, W1g1, W1g2, Wu0, Wu1, Wu2, Wu3, Wg0, Wg1, Wg2):
    i32 = jnp.int32
    cuf = (Wu1[:, None, None, :] + Wu2[None, :, None, :]
           + Wu3[None, None, :, :]).reshape(_NCU * _EMB)
    cgf = (Wg1[:, None, :] + Wg2[None, :, :]).reshape(_NCG * _EMB)
    w1g0f = W1g0[:, 0]
    w1cf = (W1g1[:, 0][:, None] + W1g2[:, 0][None, :]).reshape(_NCG)

    out = _build_fm()(
        gid.astype(i32), pubtime.astype(i32), category.astype(i32),
        uid.astype(i32), job.astype(i32), sex.astype(i32), age.astype(i32),
        Wu0, Wg0, cuf, cgf, w1g0f, w1cf)
    return out[:, None]
